# baseline (speedup 1.0000x reference)
import functools

import jax
import jax.numpy as jnp
from jax import lax
from jax.experimental import pallas as pl
from jax.experimental.pallas import tpu as pltpu
from jax.experimental.pallas import tpu_sc as plsc

VOCAB = 1000000
DIM = 64
BATCH = 16384

NC = 2
NS = 16
NW = NC * NS
B_PER_W = BATCH // NW
WAVE = 128
NWAVE = B_PER_W // WAVE


def _sc_gather(user, item, e2u, e2i):
  mesh = plsc.VectorSubcoreMesh(
      core_axis_name="c", subcore_axis_name="s",
      num_cores=NC, num_subcores=NS)

  @functools.partial(
      pl.kernel,
      out_type=(
          jax.ShapeDtypeStruct((BATCH, 2 * DIM), jnp.float32),
          jax.ShapeDtypeStruct((BATCH, 2 * DIM), jnp.float32),
      ),
      mesh=mesh,
      scratch_types=[
          pltpu.VMEM((B_PER_W,), jnp.int32),
          pltpu.VMEM((B_PER_W,), jnp.int32),
          pltpu.VMEM((2, WAVE, 2 * DIM), jnp.float32),
          pltpu.VMEM((2, WAVE, 2 * DIM), jnp.float32),
          pltpu.SemaphoreType.DMA,
          pltpu.SemaphoreType.DMA,
          pltpu.SemaphoreType.DMA,
          pltpu.SemaphoreType.DMA,
          pltpu.SemaphoreType.DMA,
          pltpu.SemaphoreType.DMA,
          pltpu.SemaphoreType.DMA,
          pltpu.SemaphoreType.DMA,
      ],
  )
  def k(u_hbm, i_hbm, eu_hbm, ei_hbm, vu_out, vi_out,
        idx_u, idx_i, rows_u, rows_i,
        gsu0, gsu1, gsi0, gsi1, wsu0, wsu1, wsi0, wsi1):
    gsem_u = (gsu0, gsu1)
    gsem_i = (gsi0, gsi1)
    wsem_u = (wsu0, wsu1)
    wsem_i = (wsi0, wsi1)
    wid = lax.axis_index("s") * NC + lax.axis_index("c")
    base = wid * B_PER_W
    pltpu.sync_copy(u_hbm.at[pl.ds(base, B_PER_W)], idx_u)
    pltpu.sync_copy(i_hbm.at[pl.ds(base, B_PER_W)], idx_i)

    CH = 64

    def fire_chunk(idx, e_hbm, rows, slot, off0, sem):
      def gi(q, c):
        off = off0 + q * 16
        iv = lax.shift_right_logical(idx[pl.ds(off, 16)], 1)
        for j in range(16):
          dst = pl.ds((off0 % WAVE) + q * 16 + j, 1)
          pltpu.async_copy(e_hbm.at[pl.ds(iv[j], 1)],
                           rows.at[slot].at[dst], sem)
        return c
      lax.fori_loop(0, CH // 16, gi, 0)

    def drain_chunk(e_hbm, rows, slot, sem):
      pltpu.make_async_copy(e_hbm.at[pl.ds(0, CH)],
                            rows.at[slot].at[pl.ds(0, CH)], sem).wait()

    def wait_writeback(slot):
      pltpu.make_async_copy(eu_hbm.at[pl.ds(0, WAVE)],
                            vu_out.at[pl.ds(base, WAVE)], wsem_u[slot]).wait()
      pltpu.make_async_copy(ei_hbm.at[pl.ds(0, WAVE)],
                            vi_out.at[pl.ds(base, WAVE)], wsem_i[slot]).wait()

    for g in range(NWAVE):
      slot = g % 2
      if g >= 2:
        wait_writeback(slot)
      for c in range(WAVE // CH):
        off0 = g * WAVE + c * CH
        fire_chunk(idx_u, eu_hbm, rows_u, slot, off0, gsem_u[slot])
        fire_chunk(idx_i, ei_hbm, rows_i, slot, off0, gsem_i[slot])
        drain_chunk(eu_hbm, rows_u, slot, gsem_u[slot])
        drain_chunk(ei_hbm, rows_i, slot, gsem_i[slot])
      dst = pl.ds(base + g * WAVE, WAVE)
      pltpu.async_copy(rows_u.at[slot], vu_out.at[dst], wsem_u[slot])
      pltpu.async_copy(rows_i.at[slot], vi_out.at[dst], wsem_i[slot])
    wait_writeback(0)
    wait_writeback(1)

  return k(user, item, e2u, e2i)


def _tc_body(u2_ref, i2_ref, user_ref, item_ref, w1u_ref, w1i_ref, b1_ref,
             w2_ref, b2_ref, w3_ref, b3_ref, out_ref):
  up = (user_ref[...] & 1)[:, None]
  ip = (item_ref[...] & 1)[:, None]
  u = jnp.where(up == 1, u2_ref[:, DIM:], u2_ref[:, :DIM])
  v = jnp.where(ip == 1, i2_ref[:, DIM:], i2_ref[:, :DIM])
  mm = functools.partial(jnp.dot, precision=lax.Precision.HIGHEST)
  pred = jnp.sum(u * v, axis=1)
  h = jnp.maximum(
      mm(u, w1u_ref[...]) + mm(v, w1i_ref[...]) + b1_ref[...], 0.0)
  h = jnp.maximum(mm(h, w2_ref[...]) + b2_ref[...], 0.0)
  noise = jnp.maximum(mm(h, w3_ref[...]) + b3_ref[...], 0.0)
  out_ref[...] = pred + noise[:, 0]


def kernel(user, item, embed_user, embed_item, W1, b1, W2, b2, W3, b3):
  user = user.astype(jnp.int32)
  item = item.astype(jnp.int32)
  e2u = embed_user.reshape(VOCAB // 2, 2 * DIM)
  e2i = embed_item.reshape(VOCAB // 2, 2 * DIM)
  vec_u2, vec_i2 = _sc_gather(user, item, e2u, e2i)

  w1u = W1[:DIM]
  w1i = W1[DIM:]
  blk = 4096
  rep = lambda shape: pl.BlockSpec(shape, lambda i: tuple(0 for _ in shape))
  out = pl.pallas_call(
      _tc_body,
      grid=(BATCH // blk,),
      in_specs=[
          pl.BlockSpec((blk, 2 * DIM), lambda i: (i, 0)),
          pl.BlockSpec((blk, 2 * DIM), lambda i: (i, 0)),
          pl.BlockSpec((blk,), lambda i: (i,)),
          pl.BlockSpec((blk,), lambda i: (i,)),
          rep((DIM, DIM)),
          rep((DIM, DIM)),
          rep((DIM,)),
          rep((DIM, DIM)),
          rep((DIM,)),
          rep((DIM, 1)),
          rep((1,)),
      ],
      out_specs=pl.BlockSpec((blk,), lambda i: (i,)),
      out_shape=jax.ShapeDtypeStruct((BATCH,), jnp.float32),
  )(vec_u2, vec_i2, user, item, w1u, w1i, b1, W2, b2, W3, b3)
  return out


# speedup vs baseline: 1.0301x; 1.0301x over previous
import functools

import jax
import jax.numpy as jnp
from jax import lax
from jax.experimental import pallas as pl
from jax.experimental.pallas import tpu as pltpu
from jax.experimental.pallas import tpu_sc as plsc

VOCAB = 1000000
DIM = 64
BATCH = 16384

NC = 2
NS = 16
NW = NC * NS
B_PER_W = BATCH // NW
WAVE = 128
NWAVE = B_PER_W // WAVE


def _sc_gather(user, item, e2u, e2i):
  mesh = plsc.VectorSubcoreMesh(
      core_axis_name="c", subcore_axis_name="s",
      num_cores=NC, num_subcores=NS)

  @functools.partial(
      pl.kernel,
      out_type=(
          jax.ShapeDtypeStruct((BATCH, 2 * DIM), jnp.float32),
          jax.ShapeDtypeStruct((BATCH, 2 * DIM), jnp.float32),
      ),
      mesh=mesh,
      scratch_types=[
          pltpu.VMEM((B_PER_W,), jnp.int32),
          pltpu.VMEM((B_PER_W,), jnp.int32),
          pltpu.VMEM((2, WAVE, 2 * DIM), jnp.float32),
          pltpu.VMEM((2, WAVE, 2 * DIM), jnp.float32),
          pltpu.SemaphoreType.DMA,
          pltpu.SemaphoreType.DMA,
          pltpu.SemaphoreType.DMA,
          pltpu.SemaphoreType.DMA,
          pltpu.SemaphoreType.DMA,
          pltpu.SemaphoreType.DMA,
          pltpu.SemaphoreType.DMA,
          pltpu.SemaphoreType.DMA,
      ],
  )
  def k(u_hbm, i_hbm, eu_hbm, ei_hbm, vu_out, vi_out,
        idx_u, idx_i, rows_u, rows_i,
        gsu0, gsu1, gsi0, gsi1, wsu0, wsu1, wsi0, wsi1):
    gsem_u = (gsu0, gsu1)
    gsem_i = (gsi0, gsi1)
    wsem_u = (wsu0, wsu1)
    wsem_i = (wsi0, wsi1)
    wid = lax.axis_index("s") * NC + lax.axis_index("c")
    base = wid * B_PER_W
    pltpu.sync_copy(u_hbm.at[pl.ds(base, B_PER_W)], idx_u)
    pltpu.sync_copy(i_hbm.at[pl.ds(base, B_PER_W)], idx_i)

    CH = 64

    def fire_chunk(idx, e_hbm, rows, slot, off0, sem):
      def gi(q, c):
        off = off0 + q * 16
        iv = lax.shift_right_logical(idx[pl.ds(off, 16)], 1)
        for j in range(16):
          dst = pl.ds((off0 % WAVE) + q * 16 + j, 1)
          pltpu.async_copy(e_hbm.at[pl.ds(iv[j], 1)],
                           rows.at[slot].at[dst], sem)
        return c
      lax.fori_loop(0, CH // 16, gi, 0)

    def drain_chunk(e_hbm, rows, slot, sem):
      pltpu.make_async_copy(e_hbm.at[pl.ds(0, CH)],
                            rows.at[slot].at[pl.ds(0, CH)], sem).wait()

    def wait_writeback(slot):
      pltpu.make_async_copy(eu_hbm.at[pl.ds(0, WAVE)],
                            vu_out.at[pl.ds(base, WAVE)], wsem_u[slot]).wait()
      pltpu.make_async_copy(ei_hbm.at[pl.ds(0, WAVE)],
                            vi_out.at[pl.ds(base, WAVE)], wsem_i[slot]).wait()

    for g in range(NWAVE):
      slot = g % 2
      if g >= 2:
        wait_writeback(slot)
      for c in range(WAVE // CH):
        off0 = g * WAVE + c * CH
        fire_chunk(idx_u, eu_hbm, rows_u, slot, off0, gsem_u[slot])
        fire_chunk(idx_i, ei_hbm, rows_i, slot, off0, gsem_i[slot])
        drain_chunk(eu_hbm, rows_u, slot, gsem_u[slot])
        drain_chunk(ei_hbm, rows_i, slot, gsem_i[slot])
      dst = pl.ds(base + g * WAVE, WAVE)
      pltpu.async_copy(rows_u.at[slot], vu_out.at[dst], wsem_u[slot])
      pltpu.async_copy(rows_i.at[slot], vi_out.at[dst], wsem_i[slot])
    wait_writeback(0)
    wait_writeback(1)

  return k(user, item, e2u, e2i)


def _tc_body(u2_ref, i2_ref, user_ref, item_ref, w1u_ref, w1i_ref, b1_ref,
             w2_ref, b2_ref, w3_ref, b3_ref, out_ref):
  up = (user_ref[...] & 1)[:, None]
  ip = (item_ref[...] & 1)[:, None]
  u = jnp.where(up == 1, u2_ref[:, DIM:], u2_ref[:, :DIM])
  v = jnp.where(ip == 1, i2_ref[:, DIM:], i2_ref[:, :DIM])
  pred = jnp.sum(u * v, axis=1)
  h = jnp.maximum(
      u @ w1u_ref[...] + v @ w1i_ref[...] + b1_ref[...], 0.0)
  h = jnp.maximum(h @ w2_ref[...] + b2_ref[...], 0.0)
  noise = jnp.maximum(h @ w3_ref[...] + b3_ref[...], 0.0)
  out_ref[...] = pred + noise[:, 0]


def kernel(user, item, embed_user, embed_item, W1, b1, W2, b2, W3, b3):
  user = user.astype(jnp.int32)
  item = item.astype(jnp.int32)
  e2u = embed_user.reshape(VOCAB // 2, 2 * DIM)
  e2i = embed_item.reshape(VOCAB // 2, 2 * DIM)
  vec_u2, vec_i2 = _sc_gather(user, item, e2u, e2i)

  w1u = W1[:DIM]
  w1i = W1[DIM:]
  blk = 4096
  rep = lambda shape: pl.BlockSpec(shape, lambda i: tuple(0 for _ in shape))
  out = pl.pallas_call(
      _tc_body,
      grid=(BATCH // blk,),
      in_specs=[
          pl.BlockSpec((blk, 2 * DIM), lambda i: (i, 0)),
          pl.BlockSpec((blk, 2 * DIM), lambda i: (i, 0)),
          pl.BlockSpec((blk,), lambda i: (i,)),
          pl.BlockSpec((blk,), lambda i: (i,)),
          rep((DIM, DIM)),
          rep((DIM, DIM)),
          rep((DIM,)),
          rep((DIM, DIM)),
          rep((DIM,)),
          rep((DIM, 1)),
          rep((1,)),
      ],
      out_specs=pl.BlockSpec((blk,), lambda i: (i,)),
      out_shape=jax.ShapeDtypeStruct((BATCH,), jnp.float32),
  )(vec_u2, vec_i2, user, item, w1u, w1i, b1, W2, b2, W3, b3)
  return out


# speedup vs baseline: 1.0353x; 1.0050x over previous
import functools

import jax
import jax.numpy as jnp
from jax import lax
from jax.experimental import pallas as pl
from jax.experimental.pallas import tpu as pltpu
from jax.experimental.pallas import tpu_sc as plsc

VOCAB = 1000000
DIM = 64
BATCH = 16384

NC = 2
NS = 16
NW = NC * NS
B_PER_W = BATCH // NW
WAVE = 128
NWAVE = B_PER_W // WAVE
CH = 64


def _sc_gather(user, item, embed_user, embed_item):
  mesh = plsc.VectorSubcoreMesh(
      core_axis_name="c", subcore_axis_name="s",
      num_cores=NC, num_subcores=NS)

  @functools.partial(
      pl.kernel,
      out_type=(
          jax.ShapeDtypeStruct((BATCH, DIM), jnp.float32),
          jax.ShapeDtypeStruct((BATCH, DIM), jnp.float32),
      ),
      mesh=mesh,
      compiler_params=pltpu.CompilerParams(use_tc_tiling_on_sc=False),
      scratch_types=[
          pltpu.VMEM((B_PER_W,), jnp.int32),
          pltpu.VMEM((B_PER_W,), jnp.int32),
          pltpu.VMEM((2, WAVE, DIM), jnp.float32),
          pltpu.VMEM((2, WAVE, DIM), jnp.float32),
          pltpu.SemaphoreType.DMA,
          pltpu.SemaphoreType.DMA,
          pltpu.SemaphoreType.DMA,
          pltpu.SemaphoreType.DMA,
          pltpu.SemaphoreType.DMA,
          pltpu.SemaphoreType.DMA,
          pltpu.SemaphoreType.DMA,
          pltpu.SemaphoreType.DMA,
      ],
  )
  def k(u_hbm, i_hbm, eu_hbm, ei_hbm, vu_out, vi_out,
        idx_u, idx_i, rows_u, rows_i,
        gsu0, gsu1, gsi0, gsi1, wsu0, wsu1, wsi0, wsi1):
    gsem_u = (gsu0, gsu1)
    gsem_i = (gsi0, gsi1)
    wsem_u = (wsu0, wsu1)
    wsem_i = (wsi0, wsi1)
    wid = lax.axis_index("s") * NC + lax.axis_index("c")
    base = wid * B_PER_W
    pltpu.sync_copy(u_hbm.at[pl.ds(base, B_PER_W)], idx_u)
    pltpu.sync_copy(i_hbm.at[pl.ds(base, B_PER_W)], idx_i)

    def fire_chunk(idx, e_hbm, rows, slot, off0, sem):
      def gi(q, c):
        off = off0 + q * 16
        iv = idx[pl.ds(off, 16)]
        for j in range(16):
          dst = pl.ds((off0 % WAVE) + q * 16 + j, 1)
          pltpu.async_copy(e_hbm.at[pl.ds(iv[j], 1)],
                           rows.at[slot].at[dst], sem)
        return c
      lax.fori_loop(0, CH // 16, gi, 0)

    def drain_chunk(e_hbm, rows, slot, sem):
      pltpu.make_async_copy(e_hbm.at[pl.ds(0, CH)],
                            rows.at[slot].at[pl.ds(0, CH)], sem).wait()

    def wait_writeback(slot):
      pltpu.make_async_copy(eu_hbm.at[pl.ds(0, WAVE)],
                            vu_out.at[pl.ds(base, WAVE)], wsem_u[slot]).wait()
      pltpu.make_async_copy(ei_hbm.at[pl.ds(0, WAVE)],
                            vi_out.at[pl.ds(base, WAVE)], wsem_i[slot]).wait()

    for g in range(NWAVE):
      slot = g % 2
      if g >= 2:
        wait_writeback(slot)
      for c in range(WAVE // CH):
        off0 = g * WAVE + c * CH
        fire_chunk(idx_u, eu_hbm, rows_u, slot, off0, gsem_u[slot])
        fire_chunk(idx_i, ei_hbm, rows_i, slot, off0, gsem_i[slot])
        drain_chunk(eu_hbm, rows_u, slot, gsem_u[slot])
        drain_chunk(ei_hbm, rows_i, slot, gsem_i[slot])
      dst = pl.ds(base + g * WAVE, WAVE)
      pltpu.async_copy(rows_u.at[slot], vu_out.at[dst], wsem_u[slot])
      pltpu.async_copy(rows_i.at[slot], vi_out.at[dst], wsem_i[slot])
    wait_writeback(0)
    wait_writeback(1)

  return k(user, item, embed_user, embed_item)


def _tc_body(u_ref, i_ref, w1u_ref, w1i_ref, b1_ref, w2_ref, b2_ref,
             w3_ref, b3_ref, out_ref):
  u = u_ref[...]
  v = i_ref[...]
  pred = jnp.sum(u * v, axis=1)
  h = jnp.maximum(
      u @ w1u_ref[...] + v @ w1i_ref[...] + b1_ref[...], 0.0)
  h = jnp.maximum(h @ w2_ref[...] + b2_ref[...], 0.0)
  noise = jnp.maximum(h @ w3_ref[...] + b3_ref[...], 0.0)
  out_ref[...] = pred + noise[:, 0]


def kernel(user, item, embed_user, embed_item, W1, b1, W2, b2, W3, b3):
  vec_u, vec_i = _sc_gather(user.astype(jnp.int32), item.astype(jnp.int32),
                            embed_user, embed_item)

  w1u = W1[:DIM]
  w1i = W1[DIM:]
  blk = 4096
  rep = lambda shape: pl.BlockSpec(shape, lambda i: tuple(0 for _ in shape))
  out = pl.pallas_call(
      _tc_body,
      grid=(BATCH // blk,),
      in_specs=[
          pl.BlockSpec((blk, DIM), lambda i: (i, 0)),
          pl.BlockSpec((blk, DIM), lambda i: (i, 0)),
          rep((DIM, DIM)),
          rep((DIM, DIM)),
          rep((DIM,)),
          rep((DIM, DIM)),
          rep((DIM,)),
          rep((DIM, 1)),
          rep((1,)),
      ],
      out_specs=pl.BlockSpec((blk,), lambda i: (i,)),
      out_shape=jax.ShapeDtypeStruct((BATCH,), jnp.float32),
  )(vec_u, vec_i, w1u, w1i, b1, W2, b2, W3, b3)
  return out


# speedup vs baseline: 1.6325x; 1.5768x over previous
import functools

import jax
import jax.numpy as jnp
from jax import lax
from jax.experimental import pallas as pl
from jax.experimental.pallas import tpu as pltpu
from jax.experimental.pallas import tpu_sc as plsc

VOCAB = 1000000
DIM = 64
BATCH = 16384

NC = 2
NS = 16
NW = NC * NS
B_PER_W = BATCH // NW
WAVE = 128
NWAVE = B_PER_W // WAVE
CH = 64


def _sc_gather(user, item, embed_user, embed_item):
  mesh = plsc.VectorSubcoreMesh(
      core_axis_name="c", subcore_axis_name="s",
      num_cores=NC, num_subcores=NS)

  @functools.partial(
      pl.kernel,
      out_type=(
          jax.ShapeDtypeStruct((BATCH, DIM), jnp.float32),
          jax.ShapeDtypeStruct((BATCH, DIM), jnp.float32),
      ),
      mesh=mesh,
      scratch_types=[
          pltpu.VMEM((B_PER_W,), jnp.int32),
          pltpu.VMEM((B_PER_W,), jnp.int32),
          pltpu.VMEM((2, WAVE, DIM), jnp.float32),
          pltpu.VMEM((2, WAVE, DIM), jnp.float32),
          pltpu.SemaphoreType.DMA,
          pltpu.SemaphoreType.DMA,
          pltpu.SemaphoreType.DMA,
          pltpu.SemaphoreType.DMA,
          pltpu.SemaphoreType.DMA,
          pltpu.SemaphoreType.DMA,
          pltpu.SemaphoreType.DMA,
          pltpu.SemaphoreType.DMA,
      ],
  )
  def k(u_hbm, i_hbm, eu_hbm, ei_hbm, vu_out, vi_out,
        idx_u, idx_i, rows_u, rows_i,
        gsu0, gsu1, gsi0, gsi1, wsu0, wsu1, wsi0, wsi1):
    gsem_u = (gsu0, gsu1)
    gsem_i = (gsi0, gsi1)
    wsem_u = (wsu0, wsu1)
    wsem_i = (wsi0, wsi1)
    wid = lax.axis_index("s") * NC + lax.axis_index("c")
    base = wid * B_PER_W
    pltpu.sync_copy(u_hbm.at[pl.ds(base, B_PER_W)], idx_u)
    pltpu.sync_copy(i_hbm.at[pl.ds(base, B_PER_W)], idx_i)

    def fire_chunk(idx, e_hbm, rows, slot, off0, sem):
      def gi(q, c):
        off = off0 + q * 16
        iv = idx[pl.ds(off, 16)]
        for j in range(16):
          dst = pl.ds((off0 % WAVE) + q * 16 + j, 1)
          pltpu.async_copy(e_hbm.at[pl.ds(iv[j], 1)],
                           rows.at[slot].at[dst], sem)
        return c
      lax.fori_loop(0, CH // 16, gi, 0)

    def drain_chunk(e_hbm, rows, slot, sem):
      pltpu.make_async_copy(e_hbm.at[pl.ds(0, CH)],
                            rows.at[slot].at[pl.ds(0, CH)], sem).wait()

    def wait_writeback(slot):
      pltpu.make_async_copy(eu_hbm.at[pl.ds(0, WAVE)],
                            vu_out.at[pl.ds(base, WAVE)], wsem_u[slot]).wait()
      pltpu.make_async_copy(ei_hbm.at[pl.ds(0, WAVE)],
                            vi_out.at[pl.ds(base, WAVE)], wsem_i[slot]).wait()

    for g in range(NWAVE):
      slot = g % 2
      if g >= 2:
        wait_writeback(slot)
      for c in range(WAVE // CH):
        off0 = g * WAVE + c * CH
        fire_chunk(idx_u, eu_hbm, rows_u, slot, off0, gsem_u[slot])
        fire_chunk(idx_i, ei_hbm, rows_i, slot, off0, gsem_i[slot])
        drain_chunk(eu_hbm, rows_u, slot, gsem_u[slot])
        drain_chunk(ei_hbm, rows_i, slot, gsem_i[slot])
      dst = pl.ds(base + g * WAVE, WAVE)
      pltpu.async_copy(rows_u.at[slot], vu_out.at[dst], wsem_u[slot])
      pltpu.async_copy(rows_i.at[slot], vi_out.at[dst], wsem_i[slot])
    wait_writeback(0)
    wait_writeback(1)

  return k(user, item, embed_user, embed_item)


def _tc_body(u_ref, i_ref, w1u_ref, w1i_ref, b1_ref, w2_ref, b2_ref,
             w3_ref, b3_ref, out_ref):
  u = u_ref[...]
  v = i_ref[...]
  pred = jnp.sum(u * v, axis=1)
  h = jnp.maximum(
      u @ w1u_ref[...] + v @ w1i_ref[...] + b1_ref[...], 0.0)
  h = jnp.maximum(h @ w2_ref[...] + b2_ref[...], 0.0)
  noise = jnp.maximum(h @ w3_ref[...] + b3_ref[...], 0.0)
  out_ref[...] = pred + noise[:, 0]


def kernel(user, item, embed_user, embed_item, W1, b1, W2, b2, W3, b3):
  vec_u, vec_i = _sc_gather(user.astype(jnp.int32), item.astype(jnp.int32),
                            embed_user, embed_item)

  w1u = W1[:DIM]
  w1i = W1[DIM:]
  blk = 4096
  rep = lambda shape: pl.BlockSpec(shape, lambda i: tuple(0 for _ in shape))
  out = pl.pallas_call(
      _tc_body,
      grid=(BATCH // blk,),
      in_specs=[
          pl.BlockSpec((blk, DIM), lambda i: (i, 0)),
          pl.BlockSpec((blk, DIM), lambda i: (i, 0)),
          rep((DIM, DIM)),
          rep((DIM, DIM)),
          rep((DIM,)),
          rep((DIM, DIM)),
          rep((DIM,)),
          rep((DIM, 1)),
          rep((1,)),
      ],
      out_specs=pl.BlockSpec((blk,), lambda i: (i,)),
      out_shape=jax.ShapeDtypeStruct((BATCH,), jnp.float32),
  )(vec_u, vec_i, w1u, w1i, b1, W2, b2, W3, b3)
  return out
